# Initial kernel scaffold; baseline (speedup 1.0000x reference)
import jax
import jax.numpy as jnp
from jax.experimental import pallas as pl


def kernel(x, W):
    raise NotImplementedError("write your pallas kernel here")


# speedup vs baseline: 1.1566x; 1.1566x over previous
import functools

import jax
import jax.numpy as jnp
from jax.experimental import pallas as pl

TOP_K = 8


def _router_body(x_ref, w_ref, vals_ref, idx_ref, imp_ref, loss_ref):
    i = pl.program_id(0)
    logits = jax.lax.dot_general(
        x_ref[...].astype(jnp.bfloat16), w_ref[...].astype(jnp.bfloat16),
        dimension_numbers=(((1,), (1,)), ((), ())),
        preferred_element_type=jnp.float32,
    )
    m = jnp.max(logits, axis=-1, keepdims=True)
    e = jnp.exp(logits - m)
    s = jnp.sum(e, axis=-1, keepdims=True)
    p = e / s

    @pl.when(i == 0)
    def _():
        imp_ref[...] = jnp.zeros_like(imp_ref)

    imp_ref[...] += jnp.sum(p, axis=0, keepdims=True)

    num_e = p.shape[-1]
    iota = jax.lax.broadcasted_iota(jnp.int32, p.shape, 1)
    cur = p
    vlist, ilist = [], []
    for _ in range(TOP_K):
        mk = jnp.max(cur, axis=-1, keepdims=True)
        ik = jnp.min(jnp.where(cur == mk, iota, num_e), axis=-1, keepdims=True)
        vlist.append(mk)
        ilist.append(ik)
        cur = jnp.where(iota == ik, -1.0, cur)
    vals_ref[...] = jnp.concatenate(vlist, axis=1)
    idx_ref[...] = jnp.concatenate(ilist, axis=1)

    @pl.when(i == pl.num_programs(0) - 1)
    def _():
        imp = imp_ref[...]
        mean = jnp.mean(imp, axis=(0, 1), keepdims=True)
        std = jnp.sqrt(jnp.mean((imp - mean) ** 2, axis=(0, 1), keepdims=True))
        loss_ref[...] = (std / (mean + 1e-6)) ** 2


@functools.partial(jax.jit, static_argnames=("block_tokens", "interpret"))
def _router(x, W, block_tokens=512, interpret=False):
    T, D = x.shape
    E = W.shape[0]
    BT = min(block_tokens, T)
    grid = (T // BT,)
    vals, idx, _, loss = pl.pallas_call(
        _router_body,
        grid=grid,
        in_specs=[
            pl.BlockSpec((BT, D), lambda i: (i, 0)),
            pl.BlockSpec((E, D), lambda i: (0, 0)),
        ],
        out_specs=[
            pl.BlockSpec((BT, TOP_K), lambda i: (i, 0)),
            pl.BlockSpec((BT, TOP_K), lambda i: (i, 0)),
            pl.BlockSpec((1, E), lambda i: (0, 0)),
            pl.BlockSpec((1, 1), lambda i: (0, 0)),
        ],
        out_shape=[
            jax.ShapeDtypeStruct((T, TOP_K), jnp.float32),
            jax.ShapeDtypeStruct((T, TOP_K), jnp.int32),
            jax.ShapeDtypeStruct((1, E), jnp.float32),
            jax.ShapeDtypeStruct((1, 1), jnp.float32),
        ],
        interpret=interpret,
    )(x, W)
    return vals, idx, loss[0, 0]


def kernel(x, W):
    return _router(x, W)


# speedup vs baseline: 1.3718x; 1.1861x over previous
import functools

import jax
import jax.numpy as jnp
from jax.experimental import pallas as pl

TOP_K = 8


def _router_body(x_ref, w_ref, vals_ref, idx_ref, imp_ref, loss_ref):
    i = pl.program_id(0)
    logits = jax.lax.dot_general(
        x_ref[...].astype(jnp.bfloat16), w_ref[...].astype(jnp.bfloat16),
        dimension_numbers=(((1,), (1,)), ((), ())),
        preferred_element_type=jnp.float32,
    )
    m = jnp.max(logits, axis=-1, keepdims=True)
    e = jnp.exp(logits - m)
    s = jnp.sum(e, axis=-1, keepdims=True)
    p = e / s

    @pl.when(i == 0)
    def _():
        imp_ref[...] = jnp.zeros_like(imp_ref)

    imp_ref[...] += jnp.sum(p, axis=0, keepdims=True)

    num_e = p.shape[-1]
    iota = jax.lax.broadcasted_iota(jnp.int32, p.shape, 1)
    cur = p
    vlist, ilist = [], []
    for _ in range(TOP_K):
        mk = jnp.max(cur, axis=-1, keepdims=True)
        ik = jnp.min(jnp.where(cur == mk, iota, num_e), axis=-1, keepdims=True)
        vlist.append(mk)
        ilist.append(ik)
        cur = jnp.where(iota == ik, -1.0, cur)
    vals_ref[...] = jnp.concatenate(vlist, axis=1)
    idx_ref[...] = jnp.concatenate(ilist, axis=1)

    @pl.when(i == pl.num_programs(0) - 1)
    def _():
        imp = imp_ref[...]
        mean = jnp.mean(imp, axis=(0, 1), keepdims=True)
        std = jnp.sqrt(jnp.mean((imp - mean) ** 2, axis=(0, 1), keepdims=True))
        loss_ref[...] = (std / (mean + 1e-6)) ** 2


@functools.partial(jax.jit, static_argnames=("block_tokens", "interpret"))
def _router(x, W, block_tokens=1024, interpret=False):
    T, D = x.shape
    E = W.shape[0]
    BT = min(block_tokens, T)
    grid = (T // BT,)
    vals, idx, _, loss = pl.pallas_call(
        _router_body,
        grid=grid,
        in_specs=[
            pl.BlockSpec((BT, D), lambda i: (i, 0)),
            pl.BlockSpec((E, D), lambda i: (0, 0)),
        ],
        out_specs=[
            pl.BlockSpec((BT, TOP_K), lambda i: (i, 0)),
            pl.BlockSpec((BT, TOP_K), lambda i: (i, 0)),
            pl.BlockSpec((1, E), lambda i: (0, 0)),
            pl.BlockSpec((1, 1), lambda i: (0, 0)),
        ],
        out_shape=[
            jax.ShapeDtypeStruct((T, TOP_K), jnp.float32),
            jax.ShapeDtypeStruct((T, TOP_K), jnp.int32),
            jax.ShapeDtypeStruct((1, E), jnp.float32),
            jax.ShapeDtypeStruct((1, 1), jnp.float32),
        ],
        interpret=interpret,
    )(x, W)
    return vals, idx, loss[0, 0]


def kernel(x, W):
    return _router(x, W)


# speedup vs baseline: 1.6153x; 1.1776x over previous
import functools

import jax
import jax.numpy as jnp
from jax.experimental import pallas as pl

TOP_K = 8


def _router_body(x_ref, w_ref, vals_ref, idx_ref, imp_ref, loss_ref):
    i = pl.program_id(0)
    logits = jax.lax.dot_general(
        x_ref[...], w_ref[...],
        dimension_numbers=(((1,), (1,)), ((), ())),
        preferred_element_type=jnp.float32,
    )
    m = jnp.max(logits, axis=-1, keepdims=True)
    e = jnp.exp(logits - m)
    s = jnp.sum(e, axis=-1, keepdims=True)
    p = e / s

    @pl.when(i == 0)
    def _():
        imp_ref[...] = jnp.zeros_like(imp_ref)

    imp_ref[...] += jnp.sum(p, axis=0, keepdims=True)

    num_e = p.shape[-1]
    iota = jax.lax.broadcasted_iota(jnp.int32, p.shape, 1)
    bits = jax.lax.bitcast_convert_type(p, jnp.int32)
    key = jax.lax.bitcast_convert_type(
        (bits & ~(num_e - 1)) | ((num_e - 1) - iota), jnp.float32)
    vlist, ilist = [], []
    for _ in range(TOP_K):
        mk = jnp.max(key, axis=-1, keepdims=True)
        mb = jax.lax.bitcast_convert_type(mk, jnp.int32)
        ilist.append((num_e - 1) - (mb & (num_e - 1)))
        vlist.append(mk)
        key = jnp.where(key == mk, -1.0, key)
    vals_ref[...] = jnp.concatenate(vlist, axis=1)
    idx_ref[...] = jnp.concatenate(ilist, axis=1)

    @pl.when(i == pl.num_programs(0) - 1)
    def _():
        imp = imp_ref[...]
        mean = jnp.mean(imp, axis=(0, 1), keepdims=True)
        std = jnp.sqrt(jnp.mean((imp - mean) ** 2, axis=(0, 1), keepdims=True))
        loss_ref[...] = (std / (mean + 1e-6)) ** 2


@functools.partial(jax.jit, static_argnames=("block_tokens", "interpret"))
def _router(x, W, block_tokens=1024, interpret=False):
    T, D = x.shape
    E = W.shape[0]
    BT = min(block_tokens, T)
    grid = (T // BT,)
    vals, idx, _, loss = pl.pallas_call(
        _router_body,
        grid=grid,
        in_specs=[
            pl.BlockSpec((BT, D), lambda i: (i, 0)),
            pl.BlockSpec((E, D), lambda i: (0, 0)),
        ],
        out_specs=[
            pl.BlockSpec((BT, TOP_K), lambda i: (i, 0)),
            pl.BlockSpec((BT, TOP_K), lambda i: (i, 0)),
            pl.BlockSpec((1, E), lambda i: (0, 0)),
            pl.BlockSpec((1, 1), lambda i: (0, 0)),
        ],
        out_shape=[
            jax.ShapeDtypeStruct((T, TOP_K), jnp.float32),
            jax.ShapeDtypeStruct((T, TOP_K), jnp.int32),
            jax.ShapeDtypeStruct((1, E), jnp.float32),
            jax.ShapeDtypeStruct((1, 1), jnp.float32),
        ],
        interpret=interpret,
    )(x, W)
    return vals, idx, loss[0, 0]


def kernel(x, W):
    return _router(x, W)
